# baseline (speedup 1.0000x reference)
import functools

import jax
import jax.numpy as jnp
from jax import lax
from jax.experimental import pallas as pl
from jax.experimental.pallas import tpu as pltpu


def _ceil_to(x, m):
    return ((x + m - 1) // m) * m


def _loss_kernel(e_hbm, m_hbm, a_ref, out_ref, e_ref, m_ref, et_ref, g_ref,
                 sem_em, *, d_p, kb):
    j = pl.program_id(0)
    n_p = e_ref.shape[0]

    @pl.when(j == 0)
    def _():
        cp_e = pltpu.make_async_copy(e_hbm, e_ref, sem_em.at[0])
        cp_m = pltpu.make_async_copy(m_hbm, m_ref, sem_em.at[1])
        cp_e.start()
        cp_m.start()
        cp_e.wait()
        e = e_ref[...]
        sq = e * e
        nrm2 = jnp.dot(sq, jnp.ones((d_p, 128), jnp.float32),
                       preferred_element_type=jnp.float32)
        inv = lax.rsqrt(jnp.maximum(nrm2, 1e-24))
        e_nbf = (e * inv).astype(jnp.bfloat16)
        e_ext = jnp.concatenate(
            [e_nbf, jnp.ones((n_p, 8), jnp.bfloat16)], axis=1)
        et = e_ext.T
        et_ref[...] = et
        g_ref[...] = jnp.dot(et[0:d_p, :], e_nbf,
                             preferred_element_type=jnp.float32)
        out_ref[...] = jnp.zeros_like(out_ref)
        cp_m.wait()

    et = et_ref[...]
    ent = et[0:d_p, :]
    g = g_ref[...]

    val = jnp.float32(0.0)
    for kk in range(kb):
        a = a_ref[kk].astype(jnp.bfloat16)
        c = jnp.dot(et, a, preferred_element_type=jnp.float32)
        sum_a = jnp.sum(c[d_p:d_p + 1, :])
        mk = m_ref[j * kb + kk]
        z = jnp.dot(mk.astype(jnp.bfloat16), ent,
                    preferred_element_type=jnp.float32)
        dot_bm = jnp.sum(c[0:d_p, :] * z)
        y1 = jnp.dot(g, mk, preferred_element_type=jnp.float32)
        y2 = jnp.dot(mk, g, preferred_element_type=jnp.float32)
        t3 = jnp.sum(y1 * y2)
        val = val + sum_a - 2.0 * dot_bm + t3

    out_ref[...] += val + jnp.zeros((1, 1, 128), jnp.float32)


def kernel(E, M, A):
    n, d = E.shape
    K = M.shape[0]

    n_p = _ceil_to(n, 128)
    d_p = _ceil_to(d, 128)
    kb = 4 if K % 4 == 0 else (2 if K % 2 == 0 else 1)
    gk = K // kb

    E_p = E if E.dtype == jnp.float32 else E.astype(jnp.float32)
    M_p = M if M.dtype == jnp.float32 else M.astype(jnp.float32)
    A_p = A
    if d_p != d:
        E_p = jnp.pad(E_p, ((0, 0), (0, d_p - d)))
        M_p = jnp.pad(M_p, ((0, 0), (0, d_p - d), (0, d_p - d)))
    if n_p != n:
        E_p = jnp.pad(E_p, ((0, n_p - n), (0, 0)))
        A_p = jnp.pad(A_p, ((0, 0), (0, n_p - n), (0, n_p - n)))

    out = pl.pallas_call(
        functools.partial(_loss_kernel, d_p=d_p, kb=kb),
        out_shape=jax.ShapeDtypeStruct((1, 1, 128), jnp.float32),
        grid=(gk,),
        in_specs=[
            pl.BlockSpec(memory_space=pl.ANY),
            pl.BlockSpec(memory_space=pl.ANY),
            pl.BlockSpec((kb, n_p, n_p), lambda j: (j, 0, 0)),
        ],
        out_specs=pl.BlockSpec((1, 1, 128), lambda j: (0, 0, 0)),
        scratch_shapes=[
            pltpu.VMEM((n_p, d_p), jnp.float32),
            pltpu.VMEM((K, d_p, d_p), jnp.float32),
            pltpu.VMEM((d_p + 8, n_p), jnp.bfloat16),
            pltpu.VMEM((d_p, d_p), jnp.float32),
            pltpu.SemaphoreType.DMA((2,)),
        ],
        compiler_params=pltpu.CompilerParams(
            dimension_semantics=("arbitrary",),
            vmem_limit_bytes=63 * 2 ** 20,
        ),
    )(E_p, M_p, A_p)

    return out[0, 0, 0]
